# baseline (speedup 1.0000x reference)
import functools

import jax
import jax.numpy as jnp
from jax import lax
from jax.experimental import pallas as pl
from jax.experimental.pallas import tpu as pltpu
from jax.experimental.pallas import tpu_sc as plsc

N = 10000
E = 320000
F = 128
H = 128
C = 16

NW = 32
K = 128
J = 80
JG = J + 2
NP = 10240
RPT = NP // 16
BLK = 1000


def _mm_kernel(x_ref, w_ref, o_ref):
    o_ref[...] = jnp.dot(x_ref[...], w_ref[...],
                         preferred_element_type=jnp.float32)


def _comb_mm_kernel(p_ref, b_ref, w_ref, o_ref):
    agg = p_ref[0] + p_ref[1]
    h = jnp.maximum(agg + b_ref[...], 0.0)
    o_ref[...] = jnp.dot(h, w_ref[...], preferred_element_type=jnp.float32)


def _final_kernel(p_ref, b_ref, w_ref, b3_ref, o_ref):
    agg = p_ref[0] + p_ref[1]
    h = jnp.maximum(agg + b_ref[...], 0.0)
    logits = jnp.dot(h, w_ref[...], preferred_element_type=jnp.float32)
    logits = logits + b3_ref[...]
    m = jnp.max(logits, axis=1, keepdims=True)
    shifted = logits - m
    o_ref[...] = shifted - jnp.log(
        jnp.sum(jnp.exp(shifted), axis=1, keepdims=True))


def _sc_segsum(support, idxp, zeros):
    mesh = plsc.VectorSubcoreMesh(core_axis_name="c", subcore_axis_name="s")

    @functools.partial(
        pl.kernel,
        out_type=jax.ShapeDtypeStruct((2, NP, H), jnp.float32),
        mesh=mesh,
        scratch_types=[
            pltpu.VMEM((JG, K), jnp.int32),
            pltpu.VMEM((K,), jnp.int32),
            pltpu.VMEM((K,), jnp.int32),
            pltpu.VMEM((K,), jnp.int32),
            pltpu.VMEM((K,), jnp.int32),
            pltpu.VMEM((K, H), jnp.float32),
            pltpu.VMEM((K, H), jnp.float32),
            pltpu.VMEM_SHARED((NP, H), jnp.float32),
            pltpu.SemaphoreType.DMA,
            pltpu.SemaphoreType.DMA,
        ],
    )
    def seg_kernel(zeros_hbm, support_hbm, idx_hbm, out_hbm,
                   pk, sb0, db0, sb1, db1, rows0, rows1, acc,
                   semg0, semg1):
        c = lax.axis_index("c")
        s = lax.axis_index("s")
        wid = s * 2 + c
        pltpu.sync_copy(zeros_hbm.at[pl.ds(s * RPT, RPT)],
                        acc.at[pl.ds(s * RPT, RPT)])
        pltpu.sync_copy(idx_hbm.at[wid], pk)
        plsc.subcore_barrier()

        def unpack(j, sb, db):
            for r in range(K // 16):
                v = pk[j, pl.ds(r * 16, 16)]
                sb[pl.ds(r * 16, 16)] = v & jnp.int32(0xFFFF)
                db[pl.ds(r * 16, 16)] = lax.shift_right_logical(v, 16)

        unpack(0, sb0, db0)
        pltpu.async_copy(support_hbm.at[sb0], rows0, semg0)
        unpack(1, sb1, db1)
        pltpu.async_copy(support_hbm.at[sb1], rows1, semg1)

        def body(t, carry):
            j0 = 2 * t
            j1 = j0 + 1
            pltpu.make_async_copy(support_hbm.at[sb0], rows0, semg0).wait()
            pltpu.sync_copy(rows0, acc.at[db0], add=True)
            unpack(j0 + 2, sb0, db0)
            pltpu.async_copy(support_hbm.at[sb0], rows0, semg0)
            pltpu.make_async_copy(support_hbm.at[sb1], rows1, semg1).wait()
            pltpu.sync_copy(rows1, acc.at[db1], add=True)
            unpack(j1 + 2, sb1, db1)
            pltpu.async_copy(support_hbm.at[sb1], rows1, semg1)
            return carry

        lax.fori_loop(0, J // 2, body, 0)
        pltpu.make_async_copy(support_hbm.at[sb0], rows0, semg0).wait()
        pltpu.make_async_copy(support_hbm.at[sb1], rows1, semg1).wait()
        plsc.subcore_barrier()
        pltpu.sync_copy(acc.at[pl.ds(s * RPT, RPT)],
                        out_hbm.at[c, pl.ds(s * RPT, RPT)])

    return seg_kernel(zeros, support, idxp)


def _tc_matmul(x, w):
    return pl.pallas_call(
        _mm_kernel,
        grid=(N // BLK,),
        in_specs=[
            pl.BlockSpec((BLK, F), lambda i: (i, 0)),
            pl.BlockSpec((F, H), lambda i: (0, 0)),
        ],
        out_specs=pl.BlockSpec((BLK, H), lambda i: (i, 0)),
        out_shape=jax.ShapeDtypeStruct((N, H), jnp.float32),
    )(x, w)


def _tc_comb_matmul(part, b, w):
    return pl.pallas_call(
        _comb_mm_kernel,
        grid=(N // BLK,),
        in_specs=[
            pl.BlockSpec((2, BLK, H), lambda i: (0, i, 0)),
            pl.BlockSpec((1, H), lambda i: (0, 0)),
            pl.BlockSpec((H, H), lambda i: (0, 0)),
        ],
        out_specs=pl.BlockSpec((BLK, H), lambda i: (i, 0)),
        out_shape=jax.ShapeDtypeStruct((N, H), jnp.float32),
    )(part, b, w)


def _tc_final(part, b, w, b3):
    return pl.pallas_call(
        _final_kernel,
        grid=(N // BLK,),
        in_specs=[
            pl.BlockSpec((2, BLK, H), lambda i: (0, i, 0)),
            pl.BlockSpec((1, H), lambda i: (0, 0)),
            pl.BlockSpec((H, C), lambda i: (0, 0)),
            pl.BlockSpec((1, C), lambda i: (0, 0)),
        ],
        out_specs=pl.BlockSpec((BLK, C), lambda i: (i, 0)),
        out_shape=jax.ShapeDtypeStruct((N, C), jnp.float32),
    )(part, b, w, b3)


def kernel(x, edge_index, W1, b1, W2, b2, W3, b3):
    f32 = jnp.float32
    W1p = jnp.zeros((F, H), f32).at[:, :W1.shape[1]].set(W1)
    W2p = jnp.zeros((H, H), f32).at[:W2.shape[0], :W2.shape[1]].set(W2)
    W3p = jnp.zeros((H, C), f32).at[:W3.shape[0], :].set(W3)
    b1p = jnp.zeros((1, H), f32).at[0, :b1.shape[0]].set(b1)
    b2p = jnp.zeros((1, H), f32).at[0, :b2.shape[0]].set(b2)
    b3p = b3.reshape(1, C)

    src = edge_index[0]
    dst = edge_index[1]
    packed = jnp.bitwise_or(src, jnp.left_shift(dst, 16))
    npad = NW * J * K - E
    padval = jnp.int32(N << 16)
    idxp = jnp.concatenate(
        [packed, jnp.full((npad,), padval, jnp.int32)]).reshape(NW, J, K)
    idxp = jnp.concatenate(
        [idxp, jnp.full((NW, JG - J, K), padval, jnp.int32)], axis=1)
    zeros = jnp.zeros((NP, H), f32)

    support1 = _tc_matmul(x, W1p)
    part1 = _sc_segsum(support1, idxp, zeros)
    support2 = _tc_comb_matmul(part1, b1p, W2p)
    part2 = _sc_segsum(support2, idxp, zeros)
    return _tc_final(part2, b2p, W3p, b3p)


# speedup vs baseline: 1.3631x; 1.3631x over previous
import functools

import jax
import jax.numpy as jnp
from jax import lax
from jax.experimental import pallas as pl
from jax.experimental.pallas import tpu as pltpu
from jax.experimental.pallas import tpu_sc as plsc

N = 10000
E = 320000
F = 128
H = 128
C = 16

NW = 32
K = 128
J = 80
JG = J + 2
NP = 10240
RPT = NP // 16
BLK = 1000


def _mm_kernel(x_ref, w_ref, o_ref):
    o_ref[...] = jnp.dot(x_ref[...], w_ref[...],
                         preferred_element_type=jnp.float32)


def _comb_mm_kernel(p_ref, b_ref, w_ref, o_ref):
    agg = p_ref[0] + p_ref[1]
    h = jnp.maximum(agg + b_ref[...], 0.0)
    o_ref[...] = jnp.dot(h, w_ref[...], preferred_element_type=jnp.float32)


def _final_kernel(p_ref, b_ref, w_ref, b3_ref, o_ref):
    agg = p_ref[0] + p_ref[1]
    h = jnp.maximum(agg + b_ref[...], 0.0)
    logits = jnp.dot(h, w_ref[...], preferred_element_type=jnp.float32)
    logits = logits + b3_ref[...]
    m = jnp.max(logits, axis=1, keepdims=True)
    shifted = logits - m
    o_ref[...] = shifted - jnp.log(
        jnp.sum(jnp.exp(shifted), axis=1, keepdims=True))


def _sc_segsum(support, idxp, zeros):
    mesh = plsc.VectorSubcoreMesh(core_axis_name="c", subcore_axis_name="s")

    @functools.partial(
        pl.kernel,
        out_type=jax.ShapeDtypeStruct((2, NP, H), jnp.float32),
        mesh=mesh,
        scratch_types=[
            pltpu.VMEM((JG, K), jnp.int32),
            pltpu.VMEM((K,), jnp.int32),
            pltpu.VMEM((K,), jnp.int32),
            pltpu.VMEM((K,), jnp.int32),
            pltpu.VMEM((K,), jnp.int32),
            pltpu.VMEM((K, H), jnp.float32),
            pltpu.VMEM((K, H), jnp.float32),
            pltpu.VMEM_SHARED((NP, H), jnp.float32),
            pltpu.SemaphoreType.DMA,
            pltpu.SemaphoreType.DMA,
        ],
    )
    def seg_kernel(zeros_hbm, support_hbm, idx_hbm, out_hbm,
                   pk, sb0, db0, sb1, db1, rows0, rows1, acc,
                   semg0, semg1):
        c = lax.axis_index("c")
        s = lax.axis_index("s")
        wid = s * 2 + c
        pltpu.sync_copy(zeros_hbm.at[pl.ds(s * RPT, RPT)],
                        acc.at[pl.ds(s * RPT, RPT)])
        pltpu.sync_copy(idx_hbm.at[wid], pk)
        plsc.subcore_barrier()

        def unpack(j, sb, db):
            for r in range(K // 16):
                v = pk[j, pl.ds(r * 16, 16)]
                sb[pl.ds(r * 16, 16)] = v & jnp.int32(0xFFFF)
                db[pl.ds(r * 16, 16)] = lax.shift_right_logical(v, 16)

        def body(t, carry):
            unpack(t, sb0, db0)
            pltpu.async_copy(support_hbm.at[sb0], rows0, semg0).wait()
            pltpu.sync_copy(rows0, acc.at[db0], add=True)
            return carry

        lax.fori_loop(0, J, body, 0)
        plsc.subcore_barrier()
        pltpu.sync_copy(acc.at[pl.ds(s * RPT, RPT)],
                        out_hbm.at[c, pl.ds(s * RPT, RPT)])

    return seg_kernel(zeros, support, idxp)


def _tc_matmul(x, w):
    return pl.pallas_call(
        _mm_kernel,
        grid=(N // BLK,),
        in_specs=[
            pl.BlockSpec((BLK, F), lambda i: (i, 0)),
            pl.BlockSpec((F, H), lambda i: (0, 0)),
        ],
        out_specs=pl.BlockSpec((BLK, H), lambda i: (i, 0)),
        out_shape=jax.ShapeDtypeStruct((N, H), jnp.float32),
    )(x, w)


def _tc_comb_matmul(part, b, w):
    return pl.pallas_call(
        _comb_mm_kernel,
        grid=(N // BLK,),
        in_specs=[
            pl.BlockSpec((2, BLK, H), lambda i: (0, i, 0)),
            pl.BlockSpec((1, H), lambda i: (0, 0)),
            pl.BlockSpec((H, H), lambda i: (0, 0)),
        ],
        out_specs=pl.BlockSpec((BLK, H), lambda i: (i, 0)),
        out_shape=jax.ShapeDtypeStruct((N, H), jnp.float32),
    )(part, b, w)


def _tc_final(part, b, w, b3):
    return pl.pallas_call(
        _final_kernel,
        grid=(N // BLK,),
        in_specs=[
            pl.BlockSpec((2, BLK, H), lambda i: (0, i, 0)),
            pl.BlockSpec((1, H), lambda i: (0, 0)),
            pl.BlockSpec((H, C), lambda i: (0, 0)),
            pl.BlockSpec((1, C), lambda i: (0, 0)),
        ],
        out_specs=pl.BlockSpec((BLK, C), lambda i: (i, 0)),
        out_shape=jax.ShapeDtypeStruct((N, C), jnp.float32),
    )(part, b, w, b3)


def kernel(x, edge_index, W1, b1, W2, b2, W3, b3):
    f32 = jnp.float32
    W1p = jnp.zeros((F, H), f32).at[:, :W1.shape[1]].set(W1)
    W2p = jnp.zeros((H, H), f32).at[:W2.shape[0], :W2.shape[1]].set(W2)
    W3p = jnp.zeros((H, C), f32).at[:W3.shape[0], :].set(W3)
    b1p = jnp.zeros((1, H), f32).at[0, :b1.shape[0]].set(b1)
    b2p = jnp.zeros((1, H), f32).at[0, :b2.shape[0]].set(b2)
    b3p = b3.reshape(1, C)

    src = edge_index[0]
    dst = edge_index[1]
    packed = jnp.bitwise_or(src, jnp.left_shift(dst, 16))
    npad = NW * J * K - E
    padval = jnp.int32(N << 16)
    idxp = jnp.concatenate(
        [packed, jnp.full((npad,), padval, jnp.int32)]).reshape(NW, J, K)
    idxp = jnp.concatenate(
        [idxp, jnp.full((NW, JG - J, K), padval, jnp.int32)], axis=1)
    zeros = jnp.zeros((NP, H), f32)

    support1 = _tc_matmul(x, W1p)
    part1 = _sc_segsum(support1, idxp, zeros)
    support2 = _tc_comb_matmul(part1, b1p, W2p)
    part2 = _sc_segsum(support2, idxp, zeros)
    return _tc_final(part2, b2p, W3p, b3p)


# speedup vs baseline: 1.5647x; 1.1479x over previous
import functools

import jax
import jax.numpy as jnp
from jax import lax
from jax.experimental import pallas as pl
from jax.experimental.pallas import tpu as pltpu
from jax.experimental.pallas import tpu_sc as plsc

N = 10000
E = 320000
F = 128
H = 128
C = 16

NW = 32
K = 128
J = 80
NP = 10240
RPT = NP // 16
BLK = 1000


def _mm_kernel(x_ref, w_ref, o_ref):
    o_ref[...] = jnp.dot(x_ref[...], w_ref[...],
                         preferred_element_type=jnp.float32)


def _comb_mm_kernel(p_ref, b_ref, w_ref, o_ref):
    agg = p_ref[0] + p_ref[1]
    h = jnp.maximum(agg + b_ref[...], 0.0)
    o_ref[...] = jnp.dot(h, w_ref[...], preferred_element_type=jnp.float32)


def _final_kernel(p_ref, b_ref, w_ref, b3_ref, o_ref):
    agg = p_ref[0] + p_ref[1]
    h = jnp.maximum(agg + b_ref[...], 0.0)
    logits = jnp.dot(h, w_ref[...], preferred_element_type=jnp.float32)
    logits = logits + b3_ref[...]
    m = jnp.max(logits, axis=1, keepdims=True)
    shifted = logits - m
    o_ref[...] = shifted - jnp.log(
        jnp.sum(jnp.exp(shifted), axis=1, keepdims=True))


def _sc_segsum(support, srcp, dstp, zeros):
    mesh = plsc.VectorSubcoreMesh(core_axis_name="c", subcore_axis_name="s")

    @functools.partial(
        pl.kernel,
        out_type=jax.ShapeDtypeStruct((2, NP, H), jnp.float32),
        mesh=mesh,
        scratch_types=[
            pltpu.VMEM((J, K), jnp.int32),
            pltpu.VMEM((J, K), jnp.int32),
            pltpu.VMEM((K, H), jnp.float32),
            pltpu.VMEM_SHARED((NP, H), jnp.float32),
            pltpu.SemaphoreType.DMA,
        ],
    )
    def seg_kernel(zeros_hbm, support_hbm, src_hbm, dst_hbm, out_hbm,
                   srcv, dstv, rows, acc, sem):
        c = lax.axis_index("c")
        s = lax.axis_index("s")
        wid = s * 2 + c
        pltpu.sync_copy(zeros_hbm.at[pl.ds(s * RPT, RPT)],
                        acc.at[pl.ds(s * RPT, RPT)])
        pltpu.sync_copy(src_hbm.at[wid], srcv)
        pltpu.sync_copy(dst_hbm.at[wid], dstv)
        plsc.subcore_barrier()

        def body(j, carry):
            pltpu.async_copy(support_hbm.at[srcv.at[j]], rows, sem).wait()
            pltpu.sync_copy(rows, acc.at[dstv.at[j]], add=True)
            return carry

        lax.fori_loop(0, J, body, 0)
        plsc.subcore_barrier()
        pltpu.sync_copy(acc.at[pl.ds(s * RPT, RPT)],
                        out_hbm.at[c, pl.ds(s * RPT, RPT)])

    return seg_kernel(zeros, support, srcp, dstp)


def _tc_matmul(x, w):
    return pl.pallas_call(
        _mm_kernel,
        grid=(N // BLK,),
        in_specs=[
            pl.BlockSpec((BLK, F), lambda i: (i, 0)),
            pl.BlockSpec((F, H), lambda i: (0, 0)),
        ],
        out_specs=pl.BlockSpec((BLK, H), lambda i: (i, 0)),
        out_shape=jax.ShapeDtypeStruct((N, H), jnp.float32),
    )(x, w)


def _tc_comb_matmul(part, b, w):
    return pl.pallas_call(
        _comb_mm_kernel,
        grid=(N // BLK,),
        in_specs=[
            pl.BlockSpec((2, BLK, H), lambda i: (0, i, 0)),
            pl.BlockSpec((1, H), lambda i: (0, 0)),
            pl.BlockSpec((H, H), lambda i: (0, 0)),
        ],
        out_specs=pl.BlockSpec((BLK, H), lambda i: (i, 0)),
        out_shape=jax.ShapeDtypeStruct((N, H), jnp.float32),
    )(part, b, w)


def _tc_final(part, b, w, b3):
    return pl.pallas_call(
        _final_kernel,
        grid=(N // BLK,),
        in_specs=[
            pl.BlockSpec((2, BLK, H), lambda i: (0, i, 0)),
            pl.BlockSpec((1, H), lambda i: (0, 0)),
            pl.BlockSpec((H, C), lambda i: (0, 0)),
            pl.BlockSpec((1, C), lambda i: (0, 0)),
        ],
        out_specs=pl.BlockSpec((BLK, C), lambda i: (i, 0)),
        out_shape=jax.ShapeDtypeStruct((N, C), jnp.float32),
    )(part, b, w, b3)


def kernel(x, edge_index, W1, b1, W2, b2, W3, b3):
    f32 = jnp.float32
    W1p = jnp.zeros((F, H), f32).at[:, :W1.shape[1]].set(W1)
    W2p = jnp.zeros((H, H), f32).at[:W2.shape[0], :W2.shape[1]].set(W2)
    W3p = jnp.zeros((H, C), f32).at[:W3.shape[0], :].set(W3)
    b1p = jnp.zeros((1, H), f32).at[0, :b1.shape[0]].set(b1)
    b2p = jnp.zeros((1, H), f32).at[0, :b2.shape[0]].set(b2)
    b3p = b3.reshape(1, C)

    src = edge_index[0]
    dst = edge_index[1]
    npad = NW * J * K - E
    srcp = jnp.concatenate(
        [src, jnp.zeros((npad,), jnp.int32)]).reshape(NW, J, K)
    dstp = jnp.concatenate(
        [dst, jnp.full((npad,), N, jnp.int32)]).reshape(NW, J, K)
    zeros = jnp.zeros((NP, H), f32)

    support1 = _tc_matmul(x, W1p)
    part1 = _sc_segsum(support1, srcp, dstp, zeros)
    support2 = _tc_comb_matmul(part1, b1p, W2p)
    part2 = _sc_segsum(support2, srcp, dstp, zeros)
    return _tc_final(part2, b2p, W3p, b3p)


# speedup vs baseline: 2.4032x; 1.5358x over previous
import functools

import jax
import jax.numpy as jnp
from jax import lax
from jax.experimental import pallas as pl
from jax.experimental.pallas import tpu as pltpu
from jax.experimental.pallas import tpu_sc as plsc

N = 10000
E = 320000
F = 128
H = 128
C = 16

NW = 32
K = 128
J = 80
NP = 10240
RPT = NP // 16
BLK = 1000


def _mm_kernel(x_ref, w_ref, o_ref):
    o_ref[...] = jnp.dot(x_ref[...], w_ref[...],
                         preferred_element_type=jnp.float32)


def _comb_mm_kernel(p_ref, b_ref, w_ref, o_ref):
    agg = p_ref[0] + p_ref[1]
    h = jnp.maximum(agg + b_ref[...], 0.0)
    o_ref[...] = jnp.dot(h, w_ref[...], preferred_element_type=jnp.float32)


def _final_kernel(p_ref, b_ref, w_ref, b3_ref, o_ref):
    agg = p_ref[0] + p_ref[1]
    h = jnp.maximum(agg + b_ref[...], 0.0)
    logits = jnp.dot(h, w_ref[...], preferred_element_type=jnp.float32)
    logits = logits + b3_ref[...]
    m = jnp.max(logits, axis=1, keepdims=True)
    shifted = logits - m
    o_ref[...] = shifted - jnp.log(
        jnp.sum(jnp.exp(shifted), axis=1, keepdims=True))


def _sc_segsum(support, srcp, dstp, zeros):
    mesh = plsc.VectorSubcoreMesh(core_axis_name="c", subcore_axis_name="s")

    @functools.partial(
        pl.kernel,
        out_type=jax.ShapeDtypeStruct((2, NP, H), jnp.float32),
        mesh=mesh,
        scratch_types=[
            pltpu.VMEM((J, K), jnp.int32),
            pltpu.VMEM((J, K), jnp.int32),
            pltpu.VMEM((K, H), jnp.float32),
            pltpu.VMEM_SHARED((NP, H), jnp.float32),
            pltpu.SemaphoreType.DMA,
        ],
    )
    def seg_kernel(zeros_hbm, support_hbm, src_hbm, dst_hbm, out_hbm,
                   srcv, dstv, rows, acc, sem):
        c = lax.axis_index("c")
        s = lax.axis_index("s")
        wid = s * 2 + c
        pltpu.sync_copy(zeros_hbm.at[pl.ds(s * RPT, RPT)],
                        acc.at[pl.ds(s * RPT, RPT)])
        pltpu.sync_copy(src_hbm.at[wid], srcv)
        pltpu.sync_copy(dst_hbm.at[wid], dstv)
        plsc.subcore_barrier()

        def body(j, carry):
            pltpu.async_copy(support_hbm.at[srcv.at[j]], rows, sem).wait()
            pltpu.sync_copy(rows, acc.at[dstv.at[j]], add=True)
            return carry

        lax.fori_loop(0, J, body, 0)
        plsc.subcore_barrier()
        pltpu.sync_copy(acc.at[pl.ds(s * RPT, RPT)],
                        out_hbm.at[c, pl.ds(s * RPT, RPT)])

    return seg_kernel(zeros, support, srcp, dstp)


def _tc_matmul(x, w):
    return pl.pallas_call(
        _mm_kernel,
        grid=(N // BLK,),
        in_specs=[
            pl.BlockSpec((BLK, F), lambda i: (i, 0)),
            pl.BlockSpec((F, H), lambda i: (0, 0)),
        ],
        out_specs=pl.BlockSpec((BLK, H), lambda i: (i, 0)),
        out_shape=jax.ShapeDtypeStruct((N, H), jnp.float32),
    )(x, w)


def _tc_comb_matmul(part, b, w):
    return pl.pallas_call(
        _comb_mm_kernel,
        grid=(N // BLK,),
        in_specs=[
            pl.BlockSpec((2, BLK, H), lambda i: (0, i, 0)),
            pl.BlockSpec((1, H), lambda i: (0, 0)),
            pl.BlockSpec((H, H), lambda i: (0, 0)),
        ],
        out_specs=pl.BlockSpec((BLK, H), lambda i: (i, 0)),
        out_shape=jax.ShapeDtypeStruct((N, H), jnp.float32),
    )(part, b, w)


def _tc_final(part, b, w, b3):
    return pl.pallas_call(
        _final_kernel,
        grid=(N // BLK,),
        in_specs=[
            pl.BlockSpec((2, BLK, H), lambda i: (0, i, 0)),
            pl.BlockSpec((1, H), lambda i: (0, 0)),
            pl.BlockSpec((H, C), lambda i: (0, 0)),
            pl.BlockSpec((1, C), lambda i: (0, 0)),
        ],
        out_specs=pl.BlockSpec((BLK, C), lambda i: (i, 0)),
        out_shape=jax.ShapeDtypeStruct((N, C), jnp.float32),
    )(part, b, w, b3)


def kernel(x, edge_index, W1, b1, W2, b2, W3, b3):
    f32 = jnp.float32
    W1p = jnp.zeros((F, H), f32).at[:, :W1.shape[1]].set(W1)
    W2p = jnp.zeros((H, H), f32).at[:W2.shape[0], :W2.shape[1]].set(W2)
    W3p = jnp.zeros((H, C), f32).at[:W3.shape[0], :].set(W3)
    b1p = jnp.zeros((1, H), f32).at[0, :b1.shape[0]].set(b1)
    b2p = jnp.zeros((1, H), f32).at[0, :b2.shape[0]].set(b2)
    b3p = b3.reshape(1, C)

    src = edge_index[0]
    dst = edge_index[1]
    npad = NW * J * K - E
    pad_iota = jnp.arange(npad, dtype=jnp.int32)
    srcp = jnp.concatenate(
        [src, pad_iota % 8]).reshape(NW, J, K)
    dstp = jnp.concatenate(
        [dst, N + pad_iota % (NP - N)]).reshape(NW, J, K)
    zeros = jnp.zeros((NP, H), f32)

    support1 = _tc_matmul(x, W1p)
    part1 = _sc_segsum(support1, srcp, dstp, zeros)
    support2 = _tc_comb_matmul(part1, b1p, W2p)
    part2 = _sc_segsum(support2, srcp, dstp, zeros)
    return _tc_final(part2, b2p, W3p, b3p)
